# Initial kernel scaffold; baseline (speedup 1.0000x reference)
import jax
import jax.numpy as jnp
from jax.experimental import pallas as pl


def kernel(x, edge_index, Wq1, bq1, Wk1, bk1, Wv1, bv1, Ws1, bs1, Wq2, bq2, Wk2, bk2, Wv2, bv2, Ws2, bs2, Wq3, bq3, Wk3, bk3, Wv3, bv3, Ws3, bs3):
    raise NotImplementedError("write your pallas kernel here")


# speedup vs baseline: 1.0320x; 1.0320x over previous
import functools

import jax
import jax.numpy as jnp
from jax.experimental import pallas as pl


def _proj_kernel(x_ref, w_ref, b_ref, o_ref):
    o_ref[...] = (
        jnp.dot(x_ref[...], w_ref[...], preferred_element_type=jnp.float32)
        + b_ref[...]
    )


def _project(x, wt, b, bn=1000):
    n, fi = x.shape
    fo = wt.shape[1]
    grid = (n // bn,)
    return pl.pallas_call(
        _proj_kernel,
        grid=grid,
        in_specs=[
            pl.BlockSpec((bn, fi), lambda i: (i, 0)),
            pl.BlockSpec((fi, fo), lambda i: (0, 0)),
            pl.BlockSpec((1, fo), lambda i: (0, 0)),
        ],
        out_specs=pl.BlockSpec((bn, fo), lambda i: (i, 0)),
        out_shape=jax.ShapeDtypeStruct((n, fo), jnp.float32),
    )(x, wt, b)


def _layer(x, src, dst, Wq, bq, Wk, bk, Wv, bv, Ws, bs):
    n = x.shape[0]
    c = Wq.shape[0]
    wt = jnp.concatenate([Wq, Wk, Wv, Ws], axis=0).T
    bb = jnp.concatenate([bq, bk, bv, bs])[None, :]
    if wt.shape[1] % 128:
        pad = (-wt.shape[1]) % 128
        wt = jnp.pad(wt, ((0, 0), (0, pad)))
        bb = jnp.pad(bb, ((0, 0), (0, pad)))
    qkvs = _project(x, wt, bb)
    q = qkvs[:, 0:c]
    k = qkvs[:, c:2 * c]
    v = qkvs[:, 2 * c:3 * c]
    s = qkvs[:, 3 * c:4 * c]

    alpha = jnp.sum(q[dst] * k[src], axis=-1) / jnp.sqrt(jnp.float32(c))
    amax = jax.ops.segment_max(alpha, dst, num_segments=n)
    ex = jnp.exp(alpha - amax[dst])
    den = jax.ops.segment_sum(ex, dst, num_segments=n)
    a = ex / jnp.clip(den[dst], 1e-16, None)
    out = jax.ops.segment_sum(a[:, None] * v[src], dst, num_segments=n)
    return jax.nn.relu(out + s)


def kernel(x, edge_index, Wq1, bq1, Wk1, bk1, Wv1, bv1, Ws1, bs1, Wq2, bq2, Wk2, bk2, Wv2, bv2, Ws2, bs2, Wq3, bq3, Wk3, bk3, Wv3, bv3, Ws3, bs3):
    src = edge_index[0]
    dst = edge_index[1]
    h = _layer(x, src, dst, Wq1, bq1, Wk1, bk1, Wv1, bv1, Ws1, bs1)
    h = _layer(h, src, dst, Wq2, bq2, Wk2, bk2, Wv2, bv2, Ws2, bs2)
    h = _layer(h, src, dst, Wq3, bq3, Wk3, bk3, Wv3, bv3, Ws3, bs3)
    return h


# speedup vs baseline: 3.4813x; 3.3733x over previous
import functools

import jax
import jax.numpy as jnp
from jax import lax
from jax.experimental import pallas as pl
from jax.experimental.pallas import tpu as pltpu
from jax.experimental.pallas import tpu_sc as plsc

N = 10000
NP = 10240
E = 160000
B = 64
NC = 2
NS = 16
NW = NC * NS
EP = 163840
ROWS = EP // B
PT1 = ROWS // NW
PT2 = ROWS // NS
RN = NP // NS
BN = 640

_SC_PARAMS = pltpu.CompilerParams(
    use_tc_tiling_on_sc=False, needs_layout_passes=False)

LAYER_DIMS = {1: (400, 400, 448, 112), 2: (200, 256, 256, 64),
              3: (4, 32, 32, 16)}


def _proj_body(x, wrefs, brefs, q_ref, k_ref, v_refs, s_ref, cc):
    wq_ref, wk_ref, wv_ref, ws_ref = wrefs
    bq_ref, bk_ref, bv_ref, bs_ref = brefs
    q_ref[...] = jnp.dot(x, wq_ref[...], preferred_element_type=jnp.float32) + bq_ref[...]
    k_ref[...] = jnp.dot(x, wk_ref[...], preferred_element_type=jnp.float32) + bk_ref[...]
    v = jnp.dot(x, wv_ref[...], preferred_element_type=jnp.float32) + bv_ref[...]
    for i, vr in enumerate(v_refs):
        vr[...] = v[:, i * cc:(i + 1) * cc]
    s_ref[...] = jnp.dot(x, ws_ref[...], preferred_element_type=jnp.float32) + bs_ref[...]


def _project(xin, wts, bs, csc, cv, cc):
    fused = isinstance(xin, tuple)
    nv = cv // cc
    grid = (NP // BN,)
    w_specs = [pl.BlockSpec(w.shape, lambda i: (0, 0)) for w in wts]
    b_specs = [pl.BlockSpec(b.shape, lambda i: (0, 0)) for b in bs]
    out_shapes = (
        [jax.ShapeDtypeStruct((NP, csc), jnp.float32)] * 2
        + [jax.ShapeDtypeStruct((NP, cc), jnp.float32)] * nv
        + [jax.ShapeDtypeStruct((NP, csc), jnp.float32)]
    )
    out_specs = (
        [pl.BlockSpec((BN, csc), lambda i: (i, 0))] * 2
        + [pl.BlockSpec((BN, cc), lambda i: (i, 0))] * nv
        + [pl.BlockSpec((BN, csc), lambda i: (i, 0))]
    )
    if fused:
        aggr, sp = xin
        ccp = aggr[0].shape[1]
        cscp = sp.shape[1]
        na = len(aggr)

        def body(*refs):
            a_refs = refs[:na]
            sp_ref = refs[na]
            wrefs = refs[na + 1:na + 5]
            brefs = refs[na + 5:na + 9]
            q_ref, k_ref = refs[na + 9], refs[na + 10]
            v_refs = refs[na + 11:na + 11 + nv]
            s_ref = refs[na + 11 + nv]
            cat = jnp.concatenate([a[...] for a in a_refs], axis=1)
            x = jax.nn.relu(cat[:, :cscp] + sp_ref[...])
            _proj_body(x, wrefs, brefs, q_ref, k_ref, v_refs, s_ref, cc)

        in_specs = (
            [pl.BlockSpec((BN, ccp), lambda i: (i, 0))] * na
            + [pl.BlockSpec((BN, cscp), lambda i: (i, 0))]
            + w_specs + b_specs)
        outs = pl.pallas_call(
            body, grid=grid, in_specs=in_specs, out_specs=out_specs,
            out_shape=out_shapes,
        )(*aggr, sp, *wts, *bs)
    else:
        fi = xin.shape[1]

        def body(x_ref, *refs):
            wrefs = refs[:4]
            brefs = refs[4:8]
            q_ref, k_ref = refs[8], refs[9]
            v_refs = refs[10:10 + nv]
            s_ref = refs[10 + nv]
            _proj_body(x_ref[...], wrefs, brefs, q_ref, k_ref, v_refs,
                       s_ref, cc)

        in_specs = [pl.BlockSpec((BN, fi), lambda i: (i, 0))] + w_specs + b_specs
        outs = pl.pallas_call(
            body, grid=grid, in_specs=in_specs, out_specs=out_specs,
            out_shape=out_shapes,
        )(xin, *wts, *bs)
    q, k = outs[0], outs[1]
    v_chunks = list(outs[2:2 + nv])
    s = outs[2 + nv]
    return q, k, v_chunks, s


def _combine(aggr, sp):
    csc = sp.shape[1]
    cc = aggr[0].shape[1]
    na = len(aggr)

    def body(*refs):
        a_refs = refs[:na]
        sp_ref = refs[na]
        o_ref = refs[na + 1]
        cat = jnp.concatenate([a[...] for a in a_refs], axis=1)
        o_ref[...] = jax.nn.relu(cat[:, :csc] + sp_ref[...])

    return pl.pallas_call(
        body,
        grid=(NP // BN,),
        in_specs=([pl.BlockSpec((BN, cc), lambda i: (i, 0))] * na
                  + [pl.BlockSpec((BN, csc), lambda i: (i, 0))]),
        out_specs=pl.BlockSpec((BN, csc), lambda i: (i, 0)),
        out_shape=jax.ShapeDtypeStruct((NP, csc), jnp.float32),
    )(*aggr, sp)


def _make_k1(csc, c_real):
    njs = csc // 16
    inv = 1.0 / float(c_real) ** 0.5
    mesh = plsc.VectorSubcoreMesh(core_axis_name="c", subcore_axis_name="s")

    @functools.partial(
        pl.kernel,
        out_type=[
            jax.ShapeDtypeStruct((ROWS, B), jnp.float32),
            jax.ShapeDtypeStruct((NW, 16), jnp.float32),
        ],
        mesh=mesh,
        compiler_params=_SC_PARAMS,
        scratch_types=[
            pltpu.VMEM((PT1, B), jnp.int32),
            pltpu.VMEM((PT1, B), jnp.int32),
            pltpu.VMEM((B, csc), jnp.float32),
            pltpu.VMEM((B, csc), jnp.float32),
            pltpu.VMEM((PT1, B), jnp.float32),
            pltpu.VMEM((B * 16,), jnp.float32),
            pltpu.VMEM((16,), jnp.float32),
            pltpu.SemaphoreType.DMA,
            pltpu.SemaphoreType.DMA,
        ],
    )
    def k1(q_hbm, k_hbm, src_hbm, dst_hbm, alpha_hbm, tmax_hbm,
           ib_s, ib_d, qb, kb, af, pacc, mxb, sem1, sem2):
        wid = lax.axis_index("s") * NC + lax.axis_index("c")
        r0 = wid * PT1
        pltpu.sync_copy(src_hbm.at[pl.ds(r0, PT1)], ib_s)
        pltpu.sync_copy(dst_hbm.at[pl.ds(r0, PT1)], ib_d)
        lanes = lax.iota(jnp.int32, 16)

        def blk(r, mx):
            cq = pltpu.async_copy(q_hbm.at[ib_d.at[r]], qb, sem1)
            ck = pltpu.async_copy(k_hbm.at[ib_s.at[r]], kb, sem2)
            cq.wait()
            ck.wait()

            def edge(b, carry):
                acc0 = qb[b, pl.ds(0, 16)] * kb[b, pl.ds(0, 16)]
                acc1 = qb[b, pl.ds(16, 16)] * kb[b, pl.ds(16, 16)]
                for j in range(2, njs):
                    prod = qb[b, pl.ds(j * 16, 16)] * kb[b, pl.ds(j * 16, 16)]
                    if j % 2 == 0:
                        acc0 = acc0 + prod
                    else:
                        acc1 = acc1 + prod
                pacc[pl.ds(b * 16, 16)] = acc0 + acc1
                return carry

            lax.fori_loop(0, B, edge, 0)
            for g in range(B // 16):
                acc = jnp.zeros((16,), jnp.float32)
                rbase = lanes * 16 + (256 * g)
                for j in range(16):
                    acc = acc + plsc.load_gather(pacc, [rbase + j])
                alpha_v = acc * inv
                af[r, pl.ds(16 * g, 16)] = alpha_v
                mx = jnp.maximum(mx, alpha_v)
            return mx

        mx = lax.fori_loop(0, PT1, blk, jnp.full((16,), -1e30, jnp.float32))
        mxb[...] = mx
        pltpu.sync_copy(af, alpha_hbm.at[pl.ds(r0, PT1)])
        pltpu.sync_copy(mxb, tmax_hbm.at[wid])

    return k1


def _global_max(tm):
    mv = tm[0]
    for i in range(1, NW):
        mv = jnp.maximum(mv, tm[i])
    return jnp.max(mv)


def _make_k2():
    mesh = plsc.VectorSubcoreMesh(core_axis_name="c", subcore_axis_name="s")

    @functools.partial(
        pl.kernel,
        out_type=[jax.ShapeDtypeStruct((NC, NP), jnp.float32)],
        mesh=mesh,
        compiler_params=_SC_PARAMS,
        scratch_types=[
            pltpu.VMEM_SHARED((NP,), jnp.float32),
            pltpu.VMEM((PT2, B), jnp.int32),
            pltpu.VMEM((PT2, B), jnp.float32),
            pltpu.VMEM((B,), jnp.float32),
            pltpu.VMEM((RN,), jnp.float32),
            pltpu.VMEM((NW, 16), jnp.float32),
        ],
    )
    def k2(alpha_hbm, dst_hbm, tmax_hbm, den_hbm,
           den_sh, ib_d, af, eb, db, tm):
        cid = lax.axis_index("c")
        t = lax.axis_index("s")
        r0 = t * PT2
        pltpu.sync_copy(dst_hbm.at[pl.ds(r0, PT2)], ib_d)
        pltpu.sync_copy(alpha_hbm.at[pl.ds(r0, PT2)], af)
        pltpu.sync_copy(tmax_hbm, tm)
        gmax = _global_max(tm)

        for g in range(RN // 16):
            db[pl.ds(g * 16, 16)] = jnp.zeros((16,), jnp.float32)
        pltpu.sync_copy(db, den_sh.at[pl.ds(t * RN, RN)])
        plsc.subcore_barrier()

        def p1(r, carry):
            for g in range(B // 16):
                eb[pl.ds(g * 16, 16)] = jnp.exp(
                    af[r, pl.ds(g * 16, 16)] - gmax)
            pltpu.sync_copy(eb, den_sh.at[ib_d.at[r]], add=True)
            return carry

        lax.fori_loop(0, PT2, p1, 0)
        plsc.subcore_barrier()
        pltpu.sync_copy(den_sh.at[pl.ds(t * RN, RN)], db)
        pltpu.sync_copy(db, den_hbm.at[cid, pl.ds(t * RN, RN)])

    return k2


def _make_k3(cc):
    mesh = plsc.VectorSubcoreMesh(core_axis_name="c", subcore_axis_name="s")

    @functools.partial(
        pl.kernel,
        out_type=[
            jax.ShapeDtypeStruct((NC * NP, cc), jnp.float32),
        ],
        mesh=mesh,
        compiler_params=_SC_PARAMS,
        scratch_types=[
            pltpu.VMEM_SHARED((NP, cc), jnp.float32),
            pltpu.VMEM((PT2, B), jnp.int32),
            pltpu.VMEM((PT2, B), jnp.int32),
            pltpu.VMEM((PT2, B), jnp.float32),
            pltpu.VMEM((B, cc), jnp.float32),
            pltpu.VMEM((B,), jnp.float32),
            pltpu.VMEM((NP,), jnp.float32),
            pltpu.VMEM((B, cc), jnp.float32),
            pltpu.VMEM((NW, 16), jnp.float32),
            pltpu.SemaphoreType.DMA,
        ],
    )
    def k3(alpha_hbm, vcat_hbm, src_hbm, dst_hbm, tmax_hbm, den_hbm,
           ocat_hbm,
           out_sh, ib_s, ib_d, af, vb, ab, den_v, zb, tm, sem):
        cid = lax.axis_index("c")
        t = lax.axis_index("s")
        r0 = t * PT2
        pltpu.sync_copy(src_hbm.at[pl.ds(r0, PT2)], ib_s)
        pltpu.sync_copy(dst_hbm.at[pl.ds(r0, PT2)], ib_d)
        pltpu.sync_copy(alpha_hbm.at[pl.ds(r0, PT2)], af)
        pltpu.sync_copy(tmax_hbm, tm)
        pltpu.sync_copy(den_hbm.at[cid], den_v)
        gmax = _global_max(tm)
        voff = cid * NP

        def addoff(r, carry):
            for g in range(B // 16):
                ib_s[r, pl.ds(g * 16, 16)] = (
                    ib_s[r, pl.ds(g * 16, 16)] + voff)
            return carry

        lax.fori_loop(0, PT2, addoff, 0)

        def zrow(r, carry):
            for off in range(0, cc, 16):
                zb[r, pl.ds(off, 16)] = jnp.zeros((16,), jnp.float32)
            return carry

        lax.fori_loop(0, B, zrow, 0)
        for g in range(RN // B):
            pltpu.sync_copy(zb, out_sh.at[pl.ds(t * RN + g * B, B)])
        plsc.subcore_barrier()

        def p2(r, carry):
            cv = pltpu.async_copy(vcat_hbm.at[ib_s.at[r]], vb, sem)
            for g in range(B // 16):
                idx = ib_d[r, pl.ds(g * 16, 16)]
                den = plsc.load_gather(den_v, [idx])
                ex = jnp.exp(af[r, pl.ds(g * 16, 16)] - gmax)
                ab[pl.ds(g * 16, 16)] = ex / jnp.maximum(den, 1e-16)
            cv.wait()

            def edge(b, carry2):
                av = plsc.load_gather(ab, [jnp.full((16,), b, jnp.int32)])
                for off in range(0, cc, 16):
                    vb[b, pl.ds(off, 16)] = vb[b, pl.ds(off, 16)] * av
                return carry2

            lax.fori_loop(0, B, edge, 0)
            pltpu.sync_copy(vb, out_sh.at[ib_d.at[r]], add=True)
            return carry

        lax.fori_loop(0, PT2, p2, 0)
        plsc.subcore_barrier()

        def cp(g, carry):
            pltpu.sync_copy(out_sh.at[pl.ds(t * RN + g * B, B)], zb)
            pltpu.sync_copy(zb, ocat_hbm.at[pl.ds(voff + t * RN + g * B, B)])
            return carry

        lax.fori_loop(0, RN // B, cp, 0)

    return k3


def _pad_w(W, b, fi_pad, fo_pad):
    fo, fi = W.shape
    wt = jnp.pad(W.T, ((0, fi_pad - fi), (0, fo_pad - fo)))
    bb = jnp.pad(b, (0, fo_pad - fo))[None, :]
    return wt, bb


def _layer(xin, srcr, dstr, Wq, bq, Wk, bk, Wv, bv, Ws, bs, lnum, fi_pad):
    c_real, csc, cv, cc = LAYER_DIMS[lnum]
    wts, bbs = [], []
    for W, b in ((Wq, bq), (Wk, bk), (Wv, bv), (Ws, bs)):
        fo_pad = cv if W is Wv else csc
        wt, bb = _pad_w(W, b, fi_pad, fo_pad)
        wts.append(wt)
        bbs.append(bb)
    q, k, v_chunks, s = _project(xin, wts, bbs, csc, cv, cc)
    alpha, tmax = _make_k1(csc, c_real)(q, k, srcr, dstr)
    (den,) = _make_k2()(alpha, dstr, tmax)
    k3 = _make_k3(cc)
    aggr = []
    for i in range(0, len(v_chunks), 2):
        vcat = jnp.concatenate([v_chunks[i], v_chunks[i + 1]], axis=0)
        (ocat,) = k3(alpha, vcat, srcr, dstr, tmax, den)
        aggr.extend([ocat[:NP], ocat[NP:]])
    return aggr, s


def kernel(x, edge_index, Wq1, bq1, Wk1, bk1, Wv1, bv1, Ws1, bs1, Wq2, bq2,
           Wk2, bk2, Wv2, bv2, Ws2, bs2, Wq3, bq3, Wk3, bk3, Wv3, bv3, Ws3,
           bs3):
    src = edge_index[0]
    dst = edge_index[1]
    srcr = jnp.pad(src, (0, EP - E)).reshape(ROWS, B)
    dstr = jnp.pad(dst, (0, EP - E), constant_values=N).reshape(ROWS, B)
    x0 = jnp.pad(x, ((0, NP - N), (0, 0)))

    aggr, s1 = _layer(x0, srcr, dstr, Wq1, bq1, Wk1, bk1, Wv1, bv1,
                      Ws1, bs1, 1, 128)
    aggr, s2 = _layer((aggr, s1), srcr, dstr, Wq2, bq2, Wk2, bk2,
                      Wv2, bv2, Ws2, bs2, 2, 400)
    aggr, s3 = _layer((aggr, s2), srcr, dstr, Wq3, bq3, Wk3, bk3,
                      Wv3, bv3, Ws3, bs3, 3, 256)
    h3 = _combine(aggr, s3)
    return h3[:N, :4]
